# baseline (speedup 1.0000x reference)
import functools

import jax
import jax.numpy as jnp
from jax import lax
from jax.experimental import pallas as pl
from jax.experimental.pallas import tpu as pltpu
from jax.experimental.pallas import tpu_sc as plsc

_NUM_CORES = 2
_NUM_SUBCORES = 16
_NW = _NUM_CORES * _NUM_SUBCORES
_LANES = 16
_TABLE_PAD = 128


def _make_sc_call(n: int, n_species: int):
  chunk = ((n + _NW - 1) // _NW + _LANES - 1) // _LANES * _LANES
  assert n >= chunk and (n - chunk) % 8 == 0

  mesh = plsc.VectorSubcoreMesh(core_axis_name="c", subcore_axis_name="s")

  @functools.partial(
      pl.kernel,
      mesh=mesh,
      compiler_params=pltpu.CompilerParams(
          needs_layout_passes=False, use_tc_tiling_on_sc=False),
      out_type=jax.ShapeDtypeStruct((n, 1), jnp.float32),
      scratch_types=[
          pltpu.VMEM((chunk,), jnp.int32),
          pltpu.VMEM((chunk,), jnp.float32),
          pltpu.VMEM((chunk, 1), jnp.float32),
          pltpu.VMEM((_TABLE_PAD,), jnp.float32),
          pltpu.VMEM((_TABLE_PAD,), jnp.float32),
          pltpu.SemaphoreType.DMA,
          pltpu.SemaphoreType.DMA,
      ],
  )
  def scale_shift(x_hbm, z_hbm, s_hbm, t_hbm, out_hbm,
                  z_v, x_v, o_v, s_v, t_v, sem_z, sem_x):
    wid = lax.axis_index("s") * _NUM_CORES + lax.axis_index("c")
    base = jnp.minimum(wid * chunk, n - chunk)

    cz = pltpu.async_copy(z_hbm.at[pl.ds(base, chunk)], z_v, sem_z)
    cx = pltpu.async_copy(x_hbm.at[pl.ds(base, chunk)], x_v, sem_x)
    pltpu.sync_copy(s_hbm, s_v.at[pl.ds(0, n_species)])
    pltpu.sync_copy(t_hbm, t_v.at[pl.ds(0, n_species)])
    cz.wait()
    cx.wait()

    lane = lax.broadcasted_iota(jnp.int32, (_LANES,), 0)
    zero = jnp.zeros((_LANES,), jnp.int32)

    @plsc.parallel_loop(0, chunk, _LANES, unroll=8)
    def body(i):
      sl = pl.ds(i, _LANES)
      idx = z_v[sl]
      s = plsc.load_gather(s_v, [idx])
      t = plsc.load_gather(t_v, [idx])
      plsc.store_scatter(o_v, [i + lane, zero], s * x_v[sl] + t)

    pltpu.sync_copy(o_v, out_hbm.at[pl.ds(base, chunk), pl.ds(0, 1)])

  return scale_shift


def kernel(x, Z, scale, shift):
  n = x.shape[0]
  return _make_sc_call(n, scale.shape[0])(
      x.reshape(-1), Z.astype(jnp.int32), scale.reshape(-1),
      shift.reshape(-1))


# speedup vs baseline: 3.5190x; 3.5190x over previous
import functools

import jax
import jax.numpy as jnp
from jax import lax
from jax.experimental import pallas as pl
from jax.experimental.pallas import tpu as pltpu
from jax.experimental.pallas import tpu_sc as plsc

_NUM_CORES = 2
_NUM_SUBCORES = 16
_NW = _NUM_CORES * _NUM_SUBCORES
_LANES = 16
_TABLE_PAD = 128


def _make_sc_call(n: int, n_species: int):
  chunk = ((n + _NW - 1) // _NW + _LANES - 1) // _LANES * _LANES
  assert n >= chunk and (n - chunk) % 8 == 0

  mesh = plsc.VectorSubcoreMesh(core_axis_name="c", subcore_axis_name="s")

  @functools.partial(
      pl.kernel,
      mesh=mesh,
      compiler_params=pltpu.CompilerParams(needs_layout_passes=False),
      out_type=jax.ShapeDtypeStruct((n,), jnp.float32),
      scratch_types=[
          pltpu.VMEM((chunk,), jnp.int32),
          pltpu.VMEM((chunk,), jnp.float32),
          pltpu.VMEM((chunk,), jnp.float32),
          pltpu.VMEM((_TABLE_PAD,), jnp.float32),
          pltpu.VMEM((_TABLE_PAD,), jnp.float32),
          pltpu.SemaphoreType.DMA,
          pltpu.SemaphoreType.DMA,
      ],
  )
  def scale_shift(x_hbm, z_hbm, s_hbm, t_hbm, out_hbm,
                  z_v, x_v, o_v, s_v, t_v, sem_in, sem_tab):
    wid = lax.axis_index("s") * _NUM_CORES + lax.axis_index("c")
    base = jnp.minimum(wid * chunk, n - chunk)

    cz = pltpu.async_copy(z_hbm.at[pl.ds(base, chunk)], z_v, sem_in)
    cx = pltpu.async_copy(x_hbm.at[pl.ds(base, chunk)], x_v, sem_in)
    cs = pltpu.async_copy(s_hbm, s_v.at[pl.ds(0, n_species)], sem_tab)
    ct = pltpu.async_copy(t_hbm, t_v.at[pl.ds(0, n_species)], sem_tab)
    cz.wait()
    cx.wait()
    cs.wait()
    ct.wait()

    @plsc.parallel_loop(0, chunk, _LANES, unroll=8)
    def body(i):
      sl = pl.ds(i, _LANES)
      idx = z_v[sl]
      s = plsc.load_gather(s_v, [idx])
      t = plsc.load_gather(t_v, [idx])
      o_v[sl] = s * x_v[sl] + t

    pltpu.sync_copy(o_v, out_hbm.at[pl.ds(base, chunk)])

  return scale_shift


def kernel(x, Z, scale, shift):
  n = x.shape[0]
  out = _make_sc_call(n, scale.shape[0])(
      x.reshape(-1), Z.astype(jnp.int32), scale.reshape(-1),
      shift.reshape(-1))
  return out.reshape(n, 1)
